# baseline (speedup 1.0000x reference)
import functools

import jax
import jax.numpy as jnp
from jax.experimental import pallas as pl

DIM = 384
HEADS = 12
DH = DIM // HEADS
WS = (4, 4, 4)
TOPK = 4
NW = 512
W3L = WS[0] * WS[1] * WS[2]
W3 = W3L + TOPK
W3P = 72
NG = 512
G = 8

_NEG = -1e30


def _ln(x, g, b):
    m = x.mean(-1, keepdims=True)
    v = ((x - m) ** 2).mean(-1, keepdims=True)
    return (x - m) / jnp.sqrt(v + 1e-6) * g + b


def _gelu(x):
    return x * 0.5 * (1.0 + jax.lax.erf(x * (2.0 ** -0.5)))


def _global_kernel(xg_ref, lng_ref, lnb_ref, qkvw_ref, qkvb_ref, projw_ref,
                   projb_ref, m1w1_ref, m1b1_ref, m1w2_ref, m1b2_ref,
                   rqw_ref, rqb_ref, rkw_ref, rkb_ref,
                   xg2_ref, sg_ref):
    x = xg_ref[:]
    g = lng_ref[:]
    b = lnb_ref[:]
    xn = _ln(x, g, b)
    qkv = xn @ qkvw_ref[:] + qkvb_ref[:]
    q = qkv[:, :DIM].reshape(NG, HEADS, DH)
    k = qkv[:, DIM:2 * DIM].reshape(NG, HEADS, DH)
    v = qkv[:, 2 * DIM:].reshape(NG, HEADS, DH)
    dn = (((2,), (2,)), ((1,), (1,)))
    scores = jax.lax.dot_general(q, k, dn,
                                 preferred_element_type=jnp.float32)
    scores = scores * (DH ** -0.5)
    aw = jax.nn.softmax(scores, axis=-1)
    dn2 = (((2,), (0,)), ((0,), (1,)))
    o = jax.lax.dot_general(aw, v, dn2,
                            preferred_element_type=jnp.float32)
    o = o.transpose(1, 0, 2).reshape(NG, DIM)
    x1 = o @ projw_ref[:] + projb_ref[:] + x
    x1n = _ln(x1, g, b)
    h = _gelu(x1n @ m1w1_ref[:] + m1b1_ref[:])
    x2 = x1 + h @ m1w2_ref[:] + m1b2_ref[:]
    xg2_ref[:] = x2

    qh = x2 @ rqw_ref[:] + rqb_ref[:]
    kh = x2 @ rkw_ref[:] + rkb_ref[:]
    logits = jax.lax.dot_general(qh, kh, (((1,), (1,)), ((), ())),
                                 preferred_element_type=jnp.float32)
    cols = jax.lax.broadcasted_iota(jnp.int32, (NW, NG), 1)
    l = logits
    for t in range(TOPK):
        m = jnp.max(l, axis=1, keepdims=True)
        is_max = l >= m
        idx = jnp.min(jnp.where(is_max, cols, NG), axis=1, keepdims=True)
        onehot = (cols == idx).astype(jnp.float32)
        l = jnp.where(cols == idx, _NEG, l)
        sg_ref[t] = onehot @ x2


def _window_kernel(sl_ref, sg_ref, lng_ref, lnb_ref, qkvw_ref, qkvb_ref,
                   wow_ref, wob_ref, m2w1_ref, m2b1_ref, m2w2_ref, m2b2_ref,
                   out_ref):
    bf = jnp.bfloat16
    f32 = jnp.float32
    g = lng_ref[:]
    b = lnb_ref[:]
    xb = sl_ref[0].reshape(4, 4, 8, 4, DIM)
    sl = xb.transpose(2, 0, 1, 3, 4).reshape(G, W3L, DIM).astype(f32)
    sg = sg_ref[:].transpose(1, 0, 2)
    pad = jnp.zeros((G, W3P - W3, DIM), f32)
    sc = jnp.concatenate([sl, sg, pad], axis=1)
    scn = _ln(sc, g, b).astype(bf)
    qkv = jax.lax.dot_general(scn.reshape(G * W3P, DIM), qkvw_ref[:],
                              (((1,), (0,)), ((), ())),
                              preferred_element_type=f32) + qkvb_ref[:]
    qkv3 = qkv.astype(bf).reshape(G, W3P, 3 * DIM)
    q3 = qkv3[:, :, :DIM]
    k3 = qkv3[:, :, DIM:2 * DIM]
    v3 = qkv3[:, :, 2 * DIM:]
    R = HEADS * W3P
    hh = jax.lax.broadcasted_iota(jnp.int32, (HEADS, W3P, DIM), 0)
    jj = jax.lax.broadcasted_iota(jnp.int32, (HEADS, W3P, DIM), 1)
    cc = jax.lax.broadcasted_iota(jnp.int32, (HEADS, W3P, DIM), 2) // DH
    sel = ((hh == cc) & (jj < W3)).astype(bf)
    kblk = (k3[:, None, :, :] * sel[None]).reshape(G, R, DIM)
    vblk = (v3[:, None, :, :] * sel[None]).reshape(G, R, DIM)
    hh2 = jax.lax.broadcasted_iota(jnp.int32, (R, HEADS), 0) // W3P
    jj2 = jax.lax.broadcasted_iota(jnp.int32, (R, HEADS), 0) % W3P
    cc2 = jax.lax.broadcasted_iota(jnp.int32, (R, HEADS), 1)
    seg = ((hh2 == cc2) & (jj2 < W3)).astype(bf)
    seg3 = jnp.broadcast_to(seg[None], (G, R, HEADS))
    vcat = jnp.concatenate([vblk, seg3], axis=2)
    s3 = jax.lax.dot_general(q3, kblk, (((2,), (2,)), ((0,), (0,))),
                             preferred_element_type=f32)
    es = jnp.exp(s3.astype(bf))
    ocat = jax.lax.dot_general(es, vcat, (((2,), (1,)), ((0,), (0,))),
                               preferred_element_type=f32)
    o_pre = ocat[:, :W3L, :DIM]
    rec = 1.0 / ocat[:, :W3L, DIM:DIM + HEADS]
    recb = jnp.broadcast_to(rec[:, :, :, None],
                            (G, W3L, HEADS, DH)).reshape(G, W3L, DIM)
    out64 = (o_pre * recb).astype(bf).reshape(G * W3L, DIM)
    l1 = (jax.lax.dot_general(out64, wow_ref[:], (((1,), (0,)), ((), ())),
                              preferred_element_type=f32)
          + wob_ref[:] + sl.reshape(G * W3L, DIM))
    l1n = _ln(l1, g, b).astype(bf)
    h = _gelu(jax.lax.dot_general(l1n, m2w1_ref[:], (((1,), (0,)), ((), ())),
                                  preferred_element_type=f32) + m2b1_ref[:])
    l2 = l1 + jax.lax.dot_general(h.astype(bf), m2w2_ref[:],
                                  (((1,), (0,)), ((), ())),
                                  preferred_element_type=f32) + m2b2_ref[:]
    lw = l2.astype(bf).reshape(G, 4, 4, 4, DIM).transpose(1, 2, 0, 3, 4)
    out_ref[:] = lw.reshape(1, 4, 4, 32, DIM)


def _row(p):
    return p.reshape(1, -1)


@functools.partial(jax.jit, static_argnames=())
def kernel(x_in, x_g_in, params):
    p = params
    bsz, C, s, h, w = x_in.shape
    gs = x_g_in.shape[2]

    xt = x_in.astype(jnp.bfloat16).transpose(0, 2, 3, 4, 1)
    xg = x_g_in.transpose(0, 2, 3, 4, 1).reshape(NG, C)

    xg2, sg = pl.pallas_call(
        _global_kernel,
        out_shape=(
            jax.ShapeDtypeStruct((NG, C), jnp.float32),
            jax.ShapeDtypeStruct((TOPK, NW, C), jnp.float32),
        ),
    )(xg, _row(p['ln_g']), _row(p['ln_b']),
      p['attn_qkv_w'], _row(p['attn_qkv_b']),
      p['attn_proj_w'], _row(p['attn_proj_b']),
      p['mlp1_w1'], _row(p['mlp1_b1']), p['mlp1_w2'], _row(p['mlp1_b2']),
      p['rq_w'], _row(p['rq_b']), p['rk_w'], _row(p['rk_b']))

    qscale = jnp.concatenate([jnp.full((C,), C ** -0.5, jnp.float32),
                              jnp.ones((2 * C,), jnp.float32)])
    gqkv_w = p['gqkv_w'] * qscale[None, :]
    gqkv_b = p['gqkv_b'] * qscale
    nsteps = NW // G
    const = lambda shape: pl.BlockSpec(shape, lambda i: tuple(0 for _ in shape))
    l_win = pl.pallas_call(
        _window_kernel,
        grid=(nsteps,),
        in_specs=[
            pl.BlockSpec((1, 4, 4, 32, C), lambda i: (0, i // 8, i % 8, 0, 0)),
            pl.BlockSpec((TOPK, G, C), lambda i: (0, i, 0)),
            const((1, C)), const((1, C)),
            const((C, 3 * C)), const((1, 3 * C)),
            const((C, C)), const((1, C)),
            const((C, 4 * C)), const((1, 4 * C)),
            const((4 * C, C)), const((1, C)),
        ],
        out_specs=pl.BlockSpec((1, 4, 4, 32, C),
                               lambda i: (0, i // 8, i % 8, 0, 0)),
        out_shape=jax.ShapeDtypeStruct((bsz, s, h, w, C), jnp.bfloat16),
    )(xt, sg, _row(p['ln_g']), _row(p['ln_b']),
      gqkv_w.astype(jnp.bfloat16), _row(gqkv_b),
      p['wo_w'].astype(jnp.bfloat16), _row(p['wo_b']),
      p['mlp2_w1'].astype(jnp.bfloat16), _row(p['mlp2_b1']),
      p['mlp2_w2'].astype(jnp.bfloat16), _row(p['mlp2_b2']))

    l_out = l_win.transpose(0, 4, 1, 2, 3).astype(jnp.float32)
    g_out = xg2.reshape(bsz, gs, gs, gs, C).transpose(0, 4, 1, 2, 3)
    return l_out, g_out


# speedup vs baseline: 1.1072x; 1.1072x over previous
import functools

import jax
import jax.numpy as jnp
from jax.experimental import pallas as pl

DIM = 384
HEADS = 12
DH = DIM // HEADS
WS = (4, 4, 4)
TOPK = 4
NW = 512
W3L = WS[0] * WS[1] * WS[2]
W3 = W3L + TOPK
W3P = 72
NG = 512
G = 8

_NEG = -1e30


def _ln(x, g, b):
    m = x.mean(-1, keepdims=True)
    v = ((x - m) ** 2).mean(-1, keepdims=True)
    return (x - m) / jnp.sqrt(v + 1e-6) * g + b


def _gelu(x):
    return x * 0.5 * (1.0 + jax.lax.erf(x * (2.0 ** -0.5)))


def _global_kernel(xg_ref, lng_ref, lnb_ref, qkvw_ref, qkvb_ref, projw_ref,
                   projb_ref, m1w1_ref, m1b1_ref, m1w2_ref, m1b2_ref,
                   rqw_ref, rqb_ref, rkw_ref, rkb_ref,
                   xg2_ref, sg_ref):
    x = xg_ref[:]
    g = lng_ref[:]
    b = lnb_ref[:]
    xn = _ln(x, g, b)
    qkv = xn @ qkvw_ref[:] + qkvb_ref[:]
    q = qkv[:, :DIM].reshape(NG, HEADS, DH)
    k = qkv[:, DIM:2 * DIM].reshape(NG, HEADS, DH)
    v = qkv[:, 2 * DIM:].reshape(NG, HEADS, DH)
    dn = (((2,), (2,)), ((1,), (1,)))
    scores = jax.lax.dot_general(q, k, dn,
                                 preferred_element_type=jnp.float32)
    scores = scores * (DH ** -0.5)
    aw = jax.nn.softmax(scores, axis=-1)
    dn2 = (((2,), (0,)), ((0,), (1,)))
    o = jax.lax.dot_general(aw, v, dn2,
                            preferred_element_type=jnp.float32)
    o = o.transpose(1, 0, 2).reshape(NG, DIM)
    x1 = o @ projw_ref[:] + projb_ref[:] + x
    x1n = _ln(x1, g, b)
    h = _gelu(x1n @ m1w1_ref[:] + m1b1_ref[:])
    x2 = x1 + h @ m1w2_ref[:] + m1b2_ref[:]
    xg2_ref[:] = x2

    qh = x2 @ rqw_ref[:] + rqb_ref[:]
    kh = x2 @ rkw_ref[:] + rkb_ref[:]
    logits = jax.lax.dot_general(qh, kh, (((1,), (1,)), ((), ())),
                                 preferred_element_type=jnp.float32)
    cols = jax.lax.broadcasted_iota(jnp.int32, (NW, NG), 1)
    l = logits
    for t in range(TOPK):
        m = jnp.max(l, axis=1, keepdims=True)
        is_max = l >= m
        idx = jnp.min(jnp.where(is_max, cols, NG), axis=1, keepdims=True)
        onehot = (cols == idx).astype(jnp.float32)
        l = jnp.where(cols == idx, _NEG, l)
        sg_ref[t] = onehot @ x2


def _window_kernel(sl_ref, sg_ref, lng_ref, lnb_ref, qkvw_ref, qkvb_ref,
                   wow_ref, wob_ref, m2w1_ref, m2b1_ref, m2w2_ref, m2b2_ref,
                   out_ref):
    bf = jnp.bfloat16
    f32 = jnp.float32
    g = lng_ref[:]
    b = lnb_ref[:]
    xb = sl_ref[0].reshape(4, 4, 8, 4, DIM)
    sl = xb.transpose(2, 0, 1, 3, 4).reshape(G, W3L, DIM).astype(f32)
    sg = sg_ref[:].transpose(1, 0, 2)
    pad = jnp.zeros((G, W3P - W3, DIM), f32)
    sc = jnp.concatenate([sl, sg, pad], axis=1)
    scn = _ln(sc, g, b).astype(bf)
    qkv = jax.lax.dot_general(scn.reshape(G * W3P, DIM), qkvw_ref[:],
                              (((1,), (0,)), ((), ())),
                              preferred_element_type=f32) + qkvb_ref[:]
    qkv3 = qkv.astype(bf).reshape(G, W3P, 3 * DIM)
    q3 = qkv3[:, :, :DIM]
    k3 = qkv3[:, :, DIM:2 * DIM]
    v3 = qkv3[:, :, 2 * DIM:]
    R = HEADS * W3P
    hh = jax.lax.broadcasted_iota(jnp.int32, (HEADS, W3P, DIM), 0)
    jj = jax.lax.broadcasted_iota(jnp.int32, (HEADS, W3P, DIM), 1)
    cc = jax.lax.broadcasted_iota(jnp.int32, (HEADS, W3P, DIM), 2) // DH
    sel = ((hh == cc) & (jj < W3)).astype(bf)
    kblk = (k3[:, None, :, :] * sel[None]).reshape(G, R, DIM)
    vblk = (v3[:, None, :, :] * sel[None]).reshape(G, R, DIM)
    hh2 = jax.lax.broadcasted_iota(jnp.int32, (R, HEADS), 0) // W3P
    jj2 = jax.lax.broadcasted_iota(jnp.int32, (R, HEADS), 0) % W3P
    cc2 = jax.lax.broadcasted_iota(jnp.int32, (R, HEADS), 1)
    seg = ((hh2 == cc2) & (jj2 < W3)).astype(bf)
    seg3 = jnp.broadcast_to(seg[None], (G, R, HEADS))
    vcat = jnp.concatenate([vblk, seg3], axis=2)
    s3 = jax.lax.dot_general(q3, kblk, (((2,), (2,)), ((0,), (0,))),
                             preferred_element_type=f32)
    es = jnp.exp(s3.astype(bf))
    ocat = jax.lax.dot_general(es, vcat, (((2,), (1,)), ((0,), (0,))),
                               preferred_element_type=f32)
    o_pre = ocat[:, :W3L, :DIM].reshape(G * W3L, DIM)
    rec = (1.0 / ocat[:, :W3L, DIM:DIM + HEADS]).astype(bf)
    hx = jax.lax.broadcasted_iota(jnp.int32, (HEADS, DIM), 0)
    cx = jax.lax.broadcasted_iota(jnp.int32, (HEADS, DIM), 1) // DH
    expand = (hx == cx).astype(bf)
    recb = jax.lax.dot_general(rec.reshape(G * W3L, HEADS), expand,
                               (((1,), (0,)), ((), ())),
                               preferred_element_type=f32)
    out64 = (o_pre * recb).astype(bf)
    l1 = (jax.lax.dot_general(out64, wow_ref[:], (((1,), (0,)), ((), ())),
                              preferred_element_type=f32)
          + wob_ref[:] + sl.reshape(G * W3L, DIM))
    l1n = _ln(l1, g, b).astype(bf)
    h = _gelu(jax.lax.dot_general(l1n, m2w1_ref[:], (((1,), (0,)), ((), ())),
                                  preferred_element_type=f32) + m2b1_ref[:])
    l2 = l1 + jax.lax.dot_general(h.astype(bf), m2w2_ref[:],
                                  (((1,), (0,)), ((), ())),
                                  preferred_element_type=f32) + m2b2_ref[:]
    lw = l2.astype(bf).reshape(G, 4, 4, 4, DIM).transpose(1, 2, 0, 3, 4)
    out_ref[:] = lw.reshape(1, 4, 4, 32, DIM)


def _row(p):
    return p.reshape(1, -1)


@functools.partial(jax.jit, static_argnames=())
def kernel(x_in, x_g_in, params):
    p = params
    bsz, C, s, h, w = x_in.shape
    gs = x_g_in.shape[2]

    xt = x_in.astype(jnp.bfloat16).transpose(0, 2, 3, 4, 1)
    xg = x_g_in.transpose(0, 2, 3, 4, 1).reshape(NG, C)

    xg2, sg = pl.pallas_call(
        _global_kernel,
        out_shape=(
            jax.ShapeDtypeStruct((NG, C), jnp.float32),
            jax.ShapeDtypeStruct((TOPK, NW, C), jnp.float32),
        ),
    )(xg, _row(p['ln_g']), _row(p['ln_b']),
      p['attn_qkv_w'], _row(p['attn_qkv_b']),
      p['attn_proj_w'], _row(p['attn_proj_b']),
      p['mlp1_w1'], _row(p['mlp1_b1']), p['mlp1_w2'], _row(p['mlp1_b2']),
      p['rq_w'], _row(p['rq_b']), p['rk_w'], _row(p['rk_b']))

    qscale = jnp.concatenate([jnp.full((C,), C ** -0.5, jnp.float32),
                              jnp.ones((2 * C,), jnp.float32)])
    gqkv_w = p['gqkv_w'] * qscale[None, :]
    gqkv_b = p['gqkv_b'] * qscale
    nsteps = NW // G
    const = lambda shape: pl.BlockSpec(shape, lambda i: tuple(0 for _ in shape))
    l_win = pl.pallas_call(
        _window_kernel,
        grid=(nsteps,),
        in_specs=[
            pl.BlockSpec((1, 4, 4, 32, C), lambda i: (0, i // 8, i % 8, 0, 0)),
            pl.BlockSpec((TOPK, G, C), lambda i: (0, i, 0)),
            const((1, C)), const((1, C)),
            const((C, 3 * C)), const((1, 3 * C)),
            const((C, C)), const((1, C)),
            const((C, 4 * C)), const((1, 4 * C)),
            const((4 * C, C)), const((1, C)),
        ],
        out_specs=pl.BlockSpec((1, 4, 4, 32, C),
                               lambda i: (0, i // 8, i % 8, 0, 0)),
        out_shape=jax.ShapeDtypeStruct((bsz, s, h, w, C), jnp.bfloat16),
    )(xt, sg, _row(p['ln_g']), _row(p['ln_b']),
      gqkv_w.astype(jnp.bfloat16), _row(gqkv_b),
      p['wo_w'].astype(jnp.bfloat16), _row(p['wo_b']),
      p['mlp2_w1'].astype(jnp.bfloat16), _row(p['mlp2_b1']),
      p['mlp2_w2'].astype(jnp.bfloat16), _row(p['mlp2_b2']))

    l_out = l_win.transpose(0, 4, 1, 2, 3).astype(jnp.float32)
    g_out = xg2.reshape(bsz, gs, gs, gs, C).transpose(0, 4, 1, 2, 3)
    return l_out, g_out


# speedup vs baseline: 1.1729x; 1.0593x over previous
import functools

import jax
import jax.numpy as jnp
from jax.experimental import pallas as pl

DIM = 384
HEADS = 12
DH = DIM // HEADS
WS = (4, 4, 4)
TOPK = 4
NW = 512
W3L = WS[0] * WS[1] * WS[2]
W3 = W3L + TOPK
W3P = 72
NG = 512
G = 16

_NEG = -1e30


def _ln(x, g, b):
    m = x.mean(-1, keepdims=True)
    v = ((x - m) ** 2).mean(-1, keepdims=True)
    return (x - m) / jnp.sqrt(v + 1e-6) * g + b


def _gelu(x):
    return x * 0.5 * (1.0 + jax.lax.erf(x * (2.0 ** -0.5)))


def _global_kernel(xg_ref, lng_ref, lnb_ref, qkvw_ref, qkvb_ref, projw_ref,
                   projb_ref, m1w1_ref, m1b1_ref, m1w2_ref, m1b2_ref,
                   rqw_ref, rqb_ref, rkw_ref, rkb_ref,
                   xg2_ref, sg_ref):
    x = xg_ref[:]
    g = lng_ref[:]
    b = lnb_ref[:]
    xn = _ln(x, g, b)
    qkv = xn @ qkvw_ref[:] + qkvb_ref[:]
    q = qkv[:, :DIM].reshape(NG, HEADS, DH)
    k = qkv[:, DIM:2 * DIM].reshape(NG, HEADS, DH)
    v = qkv[:, 2 * DIM:].reshape(NG, HEADS, DH)
    dn = (((2,), (2,)), ((1,), (1,)))
    scores = jax.lax.dot_general(q, k, dn,
                                 preferred_element_type=jnp.float32)
    scores = scores * (DH ** -0.5)
    aw = jax.nn.softmax(scores, axis=-1)
    dn2 = (((2,), (0,)), ((0,), (1,)))
    o = jax.lax.dot_general(aw, v, dn2,
                            preferred_element_type=jnp.float32)
    o = o.transpose(1, 0, 2).reshape(NG, DIM)
    x1 = o @ projw_ref[:] + projb_ref[:] + x
    x1n = _ln(x1, g, b)
    h = _gelu(x1n @ m1w1_ref[:] + m1b1_ref[:])
    x2 = x1 + h @ m1w2_ref[:] + m1b2_ref[:]
    xg2_ref[:] = x2

    qh = x2 @ rqw_ref[:] + rqb_ref[:]
    kh = x2 @ rkw_ref[:] + rkb_ref[:]
    logits = jax.lax.dot_general(qh, kh, (((1,), (1,)), ((), ())),
                                 preferred_element_type=jnp.float32)
    cols = jax.lax.broadcasted_iota(jnp.int32, (NW, NG), 1)
    l = logits
    for t in range(TOPK):
        m = jnp.max(l, axis=1, keepdims=True)
        is_max = l >= m
        idx = jnp.min(jnp.where(is_max, cols, NG), axis=1, keepdims=True)
        onehot = (cols == idx).astype(jnp.float32)
        l = jnp.where(cols == idx, _NEG, l)
        sg_ref[t] = onehot @ x2


def _window_kernel(sl_ref, sg_ref, lng_ref, lnb_ref, qkvw_ref, qkvb_ref,
                   wow_ref, wob_ref, m2w1_ref, m2b1_ref, m2w2_ref, m2b2_ref,
                   out_ref):
    bf = jnp.bfloat16
    f32 = jnp.float32
    g = lng_ref[:]
    b = lnb_ref[:]
    xb = sl_ref[0].reshape(4, 2, 4, 8, 4, DIM)
    sl = xb.transpose(1, 3, 0, 2, 4, 5).reshape(G, W3L, DIM).astype(f32)
    sg = sg_ref[:].transpose(1, 0, 2)
    pad = jnp.zeros((G, W3P - W3, DIM), f32)
    sc = jnp.concatenate([sl, sg, pad], axis=1)
    scn = _ln(sc, g, b).astype(bf)
    qkv = jax.lax.dot_general(scn.reshape(G * W3P, DIM), qkvw_ref[:],
                              (((1,), (0,)), ((), ())),
                              preferred_element_type=f32) + qkvb_ref[:]
    qkv3 = qkv.astype(bf).reshape(G, W3P, 3 * DIM)
    q3 = qkv3[:, :, :DIM]
    k3 = qkv3[:, :, DIM:2 * DIM]
    v3 = qkv3[:, :, 2 * DIM:]
    R = HEADS * W3P
    hh = jax.lax.broadcasted_iota(jnp.int32, (HEADS, W3P, DIM), 0)
    jj = jax.lax.broadcasted_iota(jnp.int32, (HEADS, W3P, DIM), 1)
    cc = jax.lax.broadcasted_iota(jnp.int32, (HEADS, W3P, DIM), 2) // DH
    sel = ((hh == cc) & (jj < W3)).astype(bf)
    kblk = (k3[:, None, :, :] * sel[None]).reshape(G, R, DIM)
    vblk = (v3[:, None, :, :] * sel[None]).reshape(G, R, DIM)
    hh2 = jax.lax.broadcasted_iota(jnp.int32, (R, HEADS), 0) // W3P
    jj2 = jax.lax.broadcasted_iota(jnp.int32, (R, HEADS), 0) % W3P
    cc2 = jax.lax.broadcasted_iota(jnp.int32, (R, HEADS), 1)
    seg = ((hh2 == cc2) & (jj2 < W3)).astype(bf)
    seg3 = jnp.broadcast_to(seg[None], (G, R, HEADS))
    vcat = jnp.concatenate([vblk, seg3], axis=2)
    s3 = jax.lax.dot_general(q3, kblk, (((2,), (2,)), ((0,), (0,))),
                             preferred_element_type=f32)
    es = jnp.exp(s3.astype(bf))
    ocat = jax.lax.dot_general(es, vcat, (((2,), (1,)), ((0,), (0,))),
                               preferred_element_type=f32)
    o_pre = ocat[:, :W3L, :DIM].reshape(G * W3L, DIM)
    rec = (1.0 / ocat[:, :W3L, DIM:DIM + HEADS]).astype(bf)
    hx = jax.lax.broadcasted_iota(jnp.int32, (HEADS, DIM), 0)
    cx = jax.lax.broadcasted_iota(jnp.int32, (HEADS, DIM), 1) // DH
    expand = (hx == cx).astype(bf)
    recb = jax.lax.dot_general(rec.reshape(G * W3L, HEADS), expand,
                               (((1,), (0,)), ((), ())),
                               preferred_element_type=f32)
    out64 = (o_pre * recb).astype(bf)
    l1 = (jax.lax.dot_general(out64, wow_ref[:], (((1,), (0,)), ((), ())),
                              preferred_element_type=f32)
          + wob_ref[:] + sl.reshape(G * W3L, DIM))
    l1n = _ln(l1, g, b).astype(bf)
    h = _gelu(jax.lax.dot_general(l1n, m2w1_ref[:], (((1,), (0,)), ((), ())),
                                  preferred_element_type=f32) + m2b1_ref[:])
    l2 = l1 + jax.lax.dot_general(h.astype(bf), m2w2_ref[:],
                                  (((1,), (0,)), ((), ())),
                                  preferred_element_type=f32) + m2b2_ref[:]
    lw = l2.astype(bf).reshape(2, 8, 4, 4, 4, DIM).transpose(2, 0, 3, 1, 4, 5)
    out_ref[:] = lw.reshape(1, 4, 8, 32, DIM)


def _row(p):
    return p.reshape(1, -1)


@functools.partial(jax.jit, static_argnames=())
def kernel(x_in, x_g_in, params):
    p = params
    bsz, C, s, h, w = x_in.shape
    gs = x_g_in.shape[2]

    xt = x_in.astype(jnp.bfloat16).transpose(0, 2, 3, 4, 1)
    xg = x_g_in.transpose(0, 2, 3, 4, 1).reshape(NG, C)

    xg2, sg = pl.pallas_call(
        _global_kernel,
        out_shape=(
            jax.ShapeDtypeStruct((NG, C), jnp.float32),
            jax.ShapeDtypeStruct((TOPK, NW, C), jnp.float32),
        ),
    )(xg, _row(p['ln_g']), _row(p['ln_b']),
      p['attn_qkv_w'], _row(p['attn_qkv_b']),
      p['attn_proj_w'], _row(p['attn_proj_b']),
      p['mlp1_w1'], _row(p['mlp1_b1']), p['mlp1_w2'], _row(p['mlp1_b2']),
      p['rq_w'], _row(p['rq_b']), p['rk_w'], _row(p['rk_b']))

    qscale = jnp.concatenate([jnp.full((C,), C ** -0.5, jnp.float32),
                              jnp.ones((2 * C,), jnp.float32)])
    gqkv_w = p['gqkv_w'] * qscale[None, :]
    gqkv_b = p['gqkv_b'] * qscale
    nsteps = NW // G
    const = lambda shape: pl.BlockSpec(shape, lambda i: tuple(0 for _ in shape))
    l_win = pl.pallas_call(
        _window_kernel,
        grid=(nsteps,),
        in_specs=[
            pl.BlockSpec((1, 4, 8, 32, C), lambda i: (0, i // 4, i % 4, 0, 0)),
            pl.BlockSpec((TOPK, G, C), lambda i: (0, i, 0)),
            const((1, C)), const((1, C)),
            const((C, 3 * C)), const((1, 3 * C)),
            const((C, C)), const((1, C)),
            const((C, 4 * C)), const((1, 4 * C)),
            const((4 * C, C)), const((1, C)),
        ],
        out_specs=pl.BlockSpec((1, 4, 8, 32, C),
                               lambda i: (0, i // 4, i % 4, 0, 0)),
        out_shape=jax.ShapeDtypeStruct((bsz, s, h, w, C), jnp.bfloat16),
    )(xt, sg, _row(p['ln_g']), _row(p['ln_b']),
      gqkv_w.astype(jnp.bfloat16), _row(gqkv_b),
      p['wo_w'].astype(jnp.bfloat16), _row(p['wo_b']),
      p['mlp2_w1'].astype(jnp.bfloat16), _row(p['mlp2_b1']),
      p['mlp2_w2'].astype(jnp.bfloat16), _row(p['mlp2_b2']))

    l_out = l_win.transpose(0, 4, 1, 2, 3).astype(jnp.float32)
    g_out = xg2.reshape(bsz, gs, gs, gs, C).transpose(0, 4, 1, 2, 3)
    return l_out, g_out
